# baseline (speedup 1.0000x reference)
import functools

import jax
import jax.numpy as jnp
from jax import lax
from jax.experimental import pallas as pl
from jax.experimental.pallas import tpu as pltpu
from jax.experimental.pallas import tpu_sc as plsc

VOCAB = 1000000
EMB = 64
HID = 256
OUT = 2
B = 4096
L = 200
NC = 2
NS = 16
LANES = 16
NW = NC * NS
BPW = B // NW
NVREG = EMB // LANES

SLAB = 512
NSLAB_FULL = VOCAB // SLAB
TAIL = VOCAB - NSLAB_FULL * SLAB


def _tr_body(tabT_hbm, out_hbm, slab_v, out_v, tail_v, tout_v):
    wid = lax.axis_index("s") * NC + lax.axis_index("c")

    iota = lax.broadcasted_iota(jnp.int32, (LANES,), 0)
    ridx = [c * LANES + iota for c in range(NVREG)]

    def do_slab(s, carry):
        c0 = s * SLAB
        pltpu.sync_copy(tabT_hbm.at[:, pl.ds(c0, SLAB)], slab_v)

        def per_tok(j, carry2):
            cidx = jnp.full((LANES,), j, jnp.int32)
            ob = pl.multiple_of(j * EMB, 8)
            for c in range(NVREG):
                vals = plsc.load_gather(slab_v, [ridx[c], cidx])
                out_v[pl.ds(ob + c * LANES, LANES)] = vals
            return carry2

        lax.fori_loop(0, SLAB, per_tok, 0)
        pltpu.sync_copy(out_v, out_hbm.at[pl.ds(c0 * EMB, SLAB * EMB)])
        return carry

    nmine = (NSLAB_FULL - 1 - wid) // NW + 1

    def loop(k, carry):
        return do_slab(wid + k * NW, carry)

    lax.fori_loop(0, nmine, loop, 0)

    @pl.when(wid == 0)
    def _():
        c0 = NSLAB_FULL * SLAB
        pltpu.sync_copy(tabT_hbm.at[:, pl.ds(c0, TAIL)], tail_v)

        def per_tok(j, carry2):
            cidx = jnp.full((LANES,), j, jnp.int32)
            ob = pl.multiple_of(j * EMB, 8)
            for c in range(NVREG):
                vals = plsc.load_gather(tail_v, [ridx[c], cidx])
                tout_v[pl.ds(ob + c * LANES, LANES)] = vals
            return carry2

        lax.fori_loop(0, TAIL, per_tok, 0)
        pltpu.sync_copy(tout_v, out_hbm.at[pl.ds(c0 * EMB, TAIL * EMB)])


def _transpose_table(tabT):
    mesh = plsc.VectorSubcoreMesh(core_axis_name="c", subcore_axis_name="s",
                                  num_cores=NC, num_subcores=NS)
    return pl.kernel(
        _tr_body,
        out_type=jax.ShapeDtypeStruct((VOCAB * EMB,), jnp.float32),
        mesh=mesh,
        scratch_types=[
            pltpu.VMEM((EMB, SLAB), jnp.float32),
            pltpu.VMEM((SLAB * EMB,), jnp.float32),
            pltpu.VMEM((EMB, TAIL), jnp.float32),
            pltpu.VMEM((TAIL * EMB,), jnp.float32),
        ],
        compiler_params=pltpu.CompilerParams(needs_layout_passes=False),
    )(tabT)


def _pool_body(x_hbm, table_hbm, out_hbm, idx_v, rows_v, out_v, sem):
    wid = lax.axis_index("s") * NC + lax.axis_index("c")
    base = wid * BPW
    pltpu.sync_copy(x_hbm.at[pl.ds(base, BPW)], idx_v)

    def per_row(b, carry):
        cp1 = pltpu.async_copy(table_hbm.at[idx_v.at[b, pl.ds(0, 104)]],
                               rows_v.at[pl.ds(0, 104)], sem)
        cp2 = pltpu.async_copy(table_hbm.at[idx_v.at[b, pl.ds(104, 96)]],
                               rows_v.at[pl.ds(104, 96)], sem)
        cp1.wait()
        cp2.wait()

        def red(l, accs):
            return tuple(accs[c] + rows_v[l, pl.ds(c * LANES, LANES)]
                         for c in range(NVREG))

        accs = lax.fori_loop(
            0, L, red,
            tuple(jnp.zeros((LANES,), jnp.float32) for _ in range(NVREG)))
        for c in range(NVREG):
            out_v[b, pl.ds(c * LANES, LANES)] = accs[c]
        return carry

    lax.fori_loop(0, BPW, per_row, 0)
    pltpu.sync_copy(out_v, out_hbm.at[pl.ds(base, BPW)])


def _pool(x, table):
    mesh = plsc.VectorSubcoreMesh(core_axis_name="c", subcore_axis_name="s",
                                  num_cores=NC, num_subcores=NS)
    return pl.kernel(
        _pool_body,
        out_type=jax.ShapeDtypeStruct((B, EMB), jnp.float32),
        mesh=mesh,
        scratch_types=[
            pltpu.VMEM((BPW, L), jnp.int32),
            pltpu.VMEM((L, EMB), jnp.float32),
            pltpu.VMEM((BPW, EMB), jnp.float32),
            pltpu.SemaphoreType.DMA,
        ],
        compiler_params=pltpu.CompilerParams(use_tc_tiling_on_sc=False),
    )(x, table)


def _mlp_body(p_ref, w1_ref, b1_ref, w2_ref, b2_ref, o_ref):
    mean = p_ref[...] * (1.0 / L)
    h1 = jnp.maximum(
        jnp.dot(mean, w1_ref[...], preferred_element_type=jnp.float32)
        + b1_ref[...], 0.0)
    logits = (jnp.dot(h1, w2_ref[...], preferred_element_type=jnp.float32)
              + b2_ref[...])
    m = jnp.max(logits, axis=1, keepdims=True)
    s = logits - m
    lse = jnp.log(jnp.sum(jnp.exp(s), axis=1, keepdims=True))
    o_ref[...] = s - lse


def _mlp(pooled, W1, b1, W2, b2):
    BLK = 512
    return pl.pallas_call(
        _mlp_body,
        grid=(B // BLK,),
        in_specs=[
            pl.BlockSpec((BLK, EMB), lambda i: (i, 0)),
            pl.BlockSpec((EMB, HID), lambda i: (0, 0)),
            pl.BlockSpec((1, HID), lambda i: (0, 0)),
            pl.BlockSpec((HID, OUT), lambda i: (0, 0)),
            pl.BlockSpec((1, OUT), lambda i: (0, 0)),
        ],
        out_specs=pl.BlockSpec((BLK, OUT), lambda i: (i, 0)),
        out_shape=jax.ShapeDtypeStruct((B, OUT), jnp.float32),
    )(pooled, W1, b1.reshape(1, HID), W2, b2.reshape(1, OUT))


def kernel(x, table, W1, b1, W2, b2):
    tlin = _transpose_table(table.T)
    pooled = _pool(x, tlin.reshape(VOCAB, EMB))
    return _mlp(pooled, W1, b1, W2, b2)


# speedup vs baseline: 1.0121x; 1.0121x over previous
import functools

import jax
import jax.numpy as jnp
from jax import lax
from jax.experimental import pallas as pl
from jax.experimental.pallas import tpu as pltpu
from jax.experimental.pallas import tpu_sc as plsc

VOCAB = 1000000
EMB = 64
HID = 256
OUT = 2
B = 4096
L = 200
NC = 2
NS = 16
LANES = 16
NW = NC * NS
BPW = B // NW
NVREG = EMB // LANES

SLAB = 512
NSLAB_FULL = VOCAB // SLAB
TAIL = VOCAB - NSLAB_FULL * SLAB


def _tr_body(tabT_hbm, out_hbm, slab_v, out_v, tail_v, tout_v):
    wid = lax.axis_index("s") * NC + lax.axis_index("c")

    iota = lax.broadcasted_iota(jnp.int32, (LANES,), 0)
    ridx = [c * LANES + iota for c in range(NVREG)]

    def do_slab(s, carry):
        c0 = s * SLAB
        pltpu.sync_copy(tabT_hbm.at[:, pl.ds(c0, SLAB)], slab_v)

        def per_tok8(jg, carry2):
            j0 = jg * 8
            ob0 = pl.multiple_of(j0 * EMB, 8)
            for dj in range(8):
                cidx = jnp.full((LANES,), j0 + dj, jnp.int32)
                for c in range(NVREG):
                    vals = plsc.load_gather(slab_v, [ridx[c], cidx])
                    out_v[pl.ds(ob0 + dj * EMB + c * LANES, LANES)] = vals
            return carry2

        lax.fori_loop(0, SLAB // 8, per_tok8, 0)
        pltpu.sync_copy(out_v, out_hbm.at[pl.ds(c0 * EMB, SLAB * EMB)])
        return carry

    nmine = (NSLAB_FULL - 1 - wid) // NW + 1

    def loop(k, carry):
        return do_slab(wid + k * NW, carry)

    lax.fori_loop(0, nmine, loop, 0)

    @pl.when(wid == 0)
    def _():
        c0 = NSLAB_FULL * SLAB
        pltpu.sync_copy(tabT_hbm.at[:, pl.ds(c0, TAIL)], tail_v)

        def per_tok8(jg, carry2):
            j0 = jg * 8
            ob0 = pl.multiple_of(j0 * EMB, 8)
            for dj in range(8):
                cidx = jnp.full((LANES,), j0 + dj, jnp.int32)
                for c in range(NVREG):
                    vals = plsc.load_gather(tail_v, [ridx[c], cidx])
                    tout_v[pl.ds(ob0 + dj * EMB + c * LANES, LANES)] = vals
            return carry2

        lax.fori_loop(0, TAIL // 8, per_tok8, 0)
        pltpu.sync_copy(tout_v, out_hbm.at[pl.ds(c0 * EMB, TAIL * EMB)])


def _transpose_table(tabT):
    mesh = plsc.VectorSubcoreMesh(core_axis_name="c", subcore_axis_name="s",
                                  num_cores=NC, num_subcores=NS)
    return pl.kernel(
        _tr_body,
        out_type=jax.ShapeDtypeStruct((VOCAB * EMB,), jnp.float32),
        mesh=mesh,
        scratch_types=[
            pltpu.VMEM((EMB, SLAB), jnp.float32),
            pltpu.VMEM((SLAB * EMB,), jnp.float32),
            pltpu.VMEM((EMB, TAIL), jnp.float32),
            pltpu.VMEM((TAIL * EMB,), jnp.float32),
        ],
        compiler_params=pltpu.CompilerParams(needs_layout_passes=False),
    )(tabT)


def _pool_body(x_hbm, table_hbm, out_hbm, idx_v, rows_v, out_v, sem):
    wid = lax.axis_index("s") * NC + lax.axis_index("c")
    base = wid * BPW
    pltpu.sync_copy(x_hbm.at[pl.ds(base, BPW)], idx_v)

    def per_row(b, carry):
        cp1 = pltpu.async_copy(table_hbm.at[idx_v.at[b, pl.ds(0, 104)]],
                               rows_v.at[pl.ds(0, 104)], sem)
        cp2 = pltpu.async_copy(table_hbm.at[idx_v.at[b, pl.ds(104, 96)]],
                               rows_v.at[pl.ds(104, 96)], sem)
        cp1.wait()
        cp2.wait()

        def red8(lg, accs):
            l0 = lg * 8
            accs = list(accs)
            for dl in range(8):
                for c in range(NVREG):
                    accs[c] = accs[c] + rows_v[l0 + dl, pl.ds(c * LANES, LANES)]
            return tuple(accs)

        accs = lax.fori_loop(
            0, L // 8, red8,
            tuple(jnp.zeros((LANES,), jnp.float32) for _ in range(NVREG)))
        for c in range(NVREG):
            out_v[b, pl.ds(c * LANES, LANES)] = accs[c]
        return carry

    lax.fori_loop(0, BPW, per_row, 0)
    pltpu.sync_copy(out_v, out_hbm.at[pl.ds(base, BPW)])


def _pool(x, table):
    mesh = plsc.VectorSubcoreMesh(core_axis_name="c", subcore_axis_name="s",
                                  num_cores=NC, num_subcores=NS)
    return pl.kernel(
        _pool_body,
        out_type=jax.ShapeDtypeStruct((B, EMB), jnp.float32),
        mesh=mesh,
        scratch_types=[
            pltpu.VMEM((BPW, L), jnp.int32),
            pltpu.VMEM((L, EMB), jnp.float32),
            pltpu.VMEM((BPW, EMB), jnp.float32),
            pltpu.SemaphoreType.DMA,
        ],
        compiler_params=pltpu.CompilerParams(use_tc_tiling_on_sc=False),
    )(x, table)


def _mlp_body(p_ref, w1_ref, b1_ref, w2_ref, b2_ref, o_ref):
    mean = p_ref[...] * (1.0 / L)
    h1 = jnp.maximum(
        jnp.dot(mean, w1_ref[...], preferred_element_type=jnp.float32)
        + b1_ref[...], 0.0)
    logits = (jnp.dot(h1, w2_ref[...], preferred_element_type=jnp.float32)
              + b2_ref[...])
    m = jnp.max(logits, axis=1, keepdims=True)
    s = logits - m
    lse = jnp.log(jnp.sum(jnp.exp(s), axis=1, keepdims=True))
    o_ref[...] = s - lse


def _mlp(pooled, W1, b1, W2, b2):
    BLK = 512
    return pl.pallas_call(
        _mlp_body,
        grid=(B // BLK,),
        in_specs=[
            pl.BlockSpec((BLK, EMB), lambda i: (i, 0)),
            pl.BlockSpec((EMB, HID), lambda i: (0, 0)),
            pl.BlockSpec((1, HID), lambda i: (0, 0)),
            pl.BlockSpec((HID, OUT), lambda i: (0, 0)),
            pl.BlockSpec((1, OUT), lambda i: (0, 0)),
        ],
        out_specs=pl.BlockSpec((BLK, OUT), lambda i: (i, 0)),
        out_shape=jax.ShapeDtypeStruct((B, OUT), jnp.float32),
    )(pooled, W1, b1.reshape(1, HID), W2, b2.reshape(1, OUT))


def kernel(x, table, W1, b1, W2, b2):
    tlin = _transpose_table(table.T)
    pooled = _pool(x, tlin.reshape(VOCAB, EMB))
    return _mlp(pooled, W1, b1, W2, b2)


# speedup vs baseline: 2.3821x; 2.3537x over previous
import functools

import jax
import jax.numpy as jnp
from jax import lax
from jax.experimental import pallas as pl
from jax.experimental.pallas import tpu as pltpu
from jax.experimental.pallas import tpu_sc as plsc

VOCAB = 1000000
EMB = 64
HID = 256
OUT = 2
B = 4096
L = 200

NC = 2
NS = 16
LANES = 16
NW = NC * NS
BPW = B // NW
NVREG = EMB // LANES


def _pool_body(x_hbm, table_hbm, out_hbm, idx_v, rows_v, out_v, sem):
    wid = lax.axis_index("s") * NC + lax.axis_index("c")
    base = wid * BPW
    pltpu.sync_copy(x_hbm.at[pl.ds(base, BPW)], idx_v)

    def per_row(b, carry):
        cp1 = pltpu.async_copy(table_hbm.at[idx_v.at[b, pl.ds(0, 104)]],
                               rows_v.at[pl.ds(0, 104)], sem)
        cp2 = pltpu.async_copy(table_hbm.at[idx_v.at[b, pl.ds(104, 96)]],
                               rows_v.at[pl.ds(104, 96)], sem)
        cp1.wait()
        cp2.wait()

        def red8(lg, accs):
            l0 = lg * 8
            accs = list(accs)
            for dl in range(8):
                for c in range(NVREG):
                    accs[c] = accs[c] + rows_v[l0 + dl, pl.ds(c * LANES, LANES)]
            return tuple(accs)

        accs = lax.fori_loop(
            0, L // 8, red8,
            tuple(jnp.zeros((LANES,), jnp.float32) for _ in range(NVREG)))
        for c in range(NVREG):
            out_v[b, pl.ds(c * LANES, LANES)] = accs[c]
        return carry

    lax.fori_loop(0, BPW, per_row, 0)
    pltpu.sync_copy(out_v, out_hbm.at[pl.ds(base, BPW)])


@functools.partial(jax.jit)
def _pool(x, table):
    mesh = plsc.VectorSubcoreMesh(core_axis_name="c", subcore_axis_name="s",
                                  num_cores=NC, num_subcores=NS)
    return pl.kernel(
        _pool_body,
        out_type=jax.ShapeDtypeStruct((B, EMB), jnp.float32),
        mesh=mesh,
        scratch_types=[
            pltpu.VMEM((BPW, L), jnp.int32),
            pltpu.VMEM((L, EMB), jnp.float32),
            pltpu.VMEM((BPW, EMB), jnp.float32),
            pltpu.SemaphoreType.DMA,
        ],
        compiler_params=pltpu.CompilerParams(use_tc_tiling_on_sc=False),
    )(x, table)


def _mlp_body(p_ref, w1_ref, b1_ref, w2_ref, b2_ref, o_ref):
    mean = p_ref[...] * (1.0 / L)
    h1 = jnp.maximum(
        jnp.dot(mean, w1_ref[...], preferred_element_type=jnp.float32)
        + b1_ref[...], 0.0)
    logits = (jnp.dot(h1, w2_ref[...], preferred_element_type=jnp.float32)
              + b2_ref[...])
    m = jnp.max(logits, axis=1, keepdims=True)
    s = logits - m
    lse = jnp.log(jnp.sum(jnp.exp(s), axis=1, keepdims=True))
    o_ref[...] = s - lse


def _mlp(pooled, W1, b1, W2, b2):
    BLK = 512
    return pl.pallas_call(
        _mlp_body,
        grid=(B // BLK,),
        in_specs=[
            pl.BlockSpec((BLK, EMB), lambda i: (i, 0)),
            pl.BlockSpec((EMB, HID), lambda i: (0, 0)),
            pl.BlockSpec((1, HID), lambda i: (0, 0)),
            pl.BlockSpec((HID, OUT), lambda i: (0, 0)),
            pl.BlockSpec((1, OUT), lambda i: (0, 0)),
        ],
        out_specs=pl.BlockSpec((BLK, OUT), lambda i: (i, 0)),
        out_shape=jax.ShapeDtypeStruct((B, OUT), jnp.float32),
    )(pooled, W1, b1.reshape(1, HID), W2, b2.reshape(1, OUT))


def kernel(x, table, W1, b1, W2, b2):
    pooled = _pool(x, table)
    return _mlp(pooled, W1, b1, W2, b2)


# speedup vs baseline: 2.8600x; 1.2006x over previous
import functools

import jax
import jax.numpy as jnp
from jax import lax
from jax.experimental import pallas as pl
from jax.experimental.pallas import tpu as pltpu
from jax.experimental.pallas import tpu_sc as plsc

VOCAB = 1000000
EMB = 64
HID = 256
OUT = 2
B = 4096
L = 200

NC = 2
NS = 16
LANES = 16
NW = NC * NS
BPW = B // NW
NVREG = EMB // LANES


DEPTH = 4


def _pool_body(x_hbm, table_hbm, out_hbm, idx_v, rows_v, out_v, *sems):
    wid = lax.axis_index("s") * NC + lax.axis_index("c")
    base = wid * BPW
    pltpu.sync_copy(x_hbm.at[pl.ds(base, BPW)], idx_v)

    def issue(b, d):
        pltpu.async_copy(table_hbm.at[idx_v.at[b, pl.ds(0, 104)]],
                         rows_v.at[d, pl.ds(0, 104)], sems[d])
        pltpu.async_copy(table_hbm.at[idx_v.at[b, pl.ds(104, 96)]],
                         rows_v.at[d, pl.ds(104, 96)], sems[d])

    def drain(d):
        pltpu.make_async_copy(table_hbm.at[pl.ds(0, 104)],
                              rows_v.at[d, pl.ds(0, 104)], sems[d]).wait()
        pltpu.make_async_copy(table_hbm.at[pl.ds(0, 96)],
                              rows_v.at[d, pl.ds(104, 96)], sems[d]).wait()

    for d in range(DEPTH):
        issue(d, d)

    def body(k, carry):
        for d in range(DEPTH):
            b = k * DEPTH + d
            drain(d)

            def red8(lg, accs):
                l0 = lg * 8
                accs = list(accs)
                for dl in range(8):
                    for c in range(NVREG):
                        accs[c] = accs[c] + rows_v[d, l0 + dl,
                                                   pl.ds(c * LANES, LANES)]
                return tuple(accs)

            accs = lax.fori_loop(
                0, L // 8, red8,
                tuple(jnp.zeros((LANES,), jnp.float32) for _ in range(NVREG)))
            for c in range(NVREG):
                out_v[b, pl.ds(c * LANES, LANES)] = accs[c]

            @pl.when(b + DEPTH < BPW)
            def _():
                issue(b + DEPTH, d)
        return carry

    lax.fori_loop(0, BPW // DEPTH, body, 0)
    pltpu.sync_copy(out_v, out_hbm.at[pl.ds(base, BPW)])


@functools.partial(jax.jit)
def _pool(x, table):
    mesh = plsc.VectorSubcoreMesh(core_axis_name="c", subcore_axis_name="s",
                                  num_cores=NC, num_subcores=NS)
    return pl.kernel(
        _pool_body,
        out_type=jax.ShapeDtypeStruct((B, EMB), jnp.float32),
        mesh=mesh,
        scratch_types=[
            pltpu.VMEM((BPW, L), jnp.int32),
            pltpu.VMEM((DEPTH, L, EMB), jnp.float32),
            pltpu.VMEM((BPW, EMB), jnp.float32),
        ] + [pltpu.SemaphoreType.DMA] * DEPTH,
        compiler_params=pltpu.CompilerParams(use_tc_tiling_on_sc=False),
    )(x, table)


def _mlp_body(p_ref, w1_ref, b1_ref, w2_ref, b2_ref, o_ref):
    mean = p_ref[...] * (1.0 / L)
    h1 = jnp.maximum(
        jnp.dot(mean, w1_ref[...], preferred_element_type=jnp.float32)
        + b1_ref[...], 0.0)
    logits = (jnp.dot(h1, w2_ref[...], preferred_element_type=jnp.float32)
              + b2_ref[...])
    m = jnp.max(logits, axis=1, keepdims=True)
    s = logits - m
    lse = jnp.log(jnp.sum(jnp.exp(s), axis=1, keepdims=True))
    o_ref[...] = s - lse


def _mlp(pooled, W1, b1, W2, b2):
    BLK = 512
    return pl.pallas_call(
        _mlp_body,
        grid=(B // BLK,),
        in_specs=[
            pl.BlockSpec((BLK, EMB), lambda i: (i, 0)),
            pl.BlockSpec((EMB, HID), lambda i: (0, 0)),
            pl.BlockSpec((1, HID), lambda i: (0, 0)),
            pl.BlockSpec((HID, OUT), lambda i: (0, 0)),
            pl.BlockSpec((1, OUT), lambda i: (0, 0)),
        ],
        out_specs=pl.BlockSpec((BLK, OUT), lambda i: (i, 0)),
        out_shape=jax.ShapeDtypeStruct((B, OUT), jnp.float32),
    )(pooled, W1, b1.reshape(1, HID), W2, b2.reshape(1, OUT))


def kernel(x, table, W1, b1, W2, b2):
    pooled = _pool(x, table)
    return _mlp(pooled, W1, b1, W2, b2)


# speedup vs baseline: 2.8645x; 1.0016x over previous
import functools

import jax
import jax.numpy as jnp
from jax import lax
from jax.experimental import pallas as pl
from jax.experimental.pallas import tpu as pltpu
from jax.experimental.pallas import tpu_sc as plsc

VOCAB = 1000000
EMB = 64
HID = 256
OUT = 2
B = 4096
L = 200

NC = 2
NS = 16
LANES = 16
NW = NC * NS
BPW = B // NW
NVREG = EMB // LANES


DEPTH = 4


def _pool_body(x_hbm, table_hbm, out_hbm, idx_v, rows_v, out_v, *sems):
    wid = lax.axis_index("s") * NC + lax.axis_index("c")
    base = wid * BPW
    pltpu.sync_copy(x_hbm.at[pl.ds(base, BPW)], idx_v)

    def issue(b, d):
        pltpu.async_copy(table_hbm.at[idx_v.at[b, pl.ds(0, 104)]],
                         rows_v.at[d, pl.ds(0, 104)], sems[d])
        pltpu.async_copy(table_hbm.at[idx_v.at[b, pl.ds(104, 96)]],
                         rows_v.at[d, pl.ds(104, 96)], sems[d])

    def drain(d):
        pltpu.make_async_copy(table_hbm.at[pl.ds(0, 104)],
                              rows_v.at[d, pl.ds(0, 104)], sems[d]).wait()
        pltpu.make_async_copy(table_hbm.at[pl.ds(0, 96)],
                              rows_v.at[d, pl.ds(104, 96)], sems[d]).wait()

    for d in range(DEPTH):
        issue(d, d)

    def body(k, carry):
        for d in range(DEPTH):
            b = k * DEPTH + d
            drain(d)

            def red8(lg, accs):
                l0 = lg * 8
                accs = list(accs)
                for dl in range(8):
                    for c in range(NVREG):
                        k_ = (dl % 2) * NVREG + c
                        accs[k_] = accs[k_] + rows_v[d, l0 + dl,
                                                     pl.ds(c * LANES, LANES)]
                return tuple(accs)

            accs = lax.fori_loop(
                0, L // 8, red8,
                tuple(jnp.zeros((LANES,), jnp.float32)
                      for _ in range(2 * NVREG)))
            for c in range(NVREG):
                out_v[b, pl.ds(c * LANES, LANES)] = accs[c] + accs[NVREG + c]

            @pl.when(b + DEPTH < BPW)
            def _():
                issue(b + DEPTH, d)
        return carry

    lax.fori_loop(0, BPW // DEPTH, body, 0)
    pltpu.sync_copy(out_v, out_hbm.at[pl.ds(base, BPW)])


@functools.partial(jax.jit)
def _pool(x, table):
    mesh = plsc.VectorSubcoreMesh(core_axis_name="c", subcore_axis_name="s",
                                  num_cores=NC, num_subcores=NS)
    return pl.kernel(
        _pool_body,
        out_type=jax.ShapeDtypeStruct((B, EMB), jnp.float32),
        mesh=mesh,
        scratch_types=[
            pltpu.VMEM((BPW, L), jnp.int32),
            pltpu.VMEM((DEPTH, L, EMB), jnp.float32),
            pltpu.VMEM((BPW, EMB), jnp.float32),
        ] + [pltpu.SemaphoreType.DMA] * DEPTH,
        compiler_params=pltpu.CompilerParams(use_tc_tiling_on_sc=False),
    )(x, table)


def _mlp_body(p_ref, w1_ref, b1_ref, w2_ref, b2_ref, o_ref):
    mean = p_ref[...] * (1.0 / L)
    h1 = jnp.maximum(
        jnp.dot(mean, w1_ref[...], preferred_element_type=jnp.float32)
        + b1_ref[...], 0.0)
    logits = (jnp.dot(h1, w2_ref[...], preferred_element_type=jnp.float32)
              + b2_ref[...])
    m = jnp.max(logits, axis=1, keepdims=True)
    s = logits - m
    lse = jnp.log(jnp.sum(jnp.exp(s), axis=1, keepdims=True))
    o_ref[...] = s - lse


def _mlp(pooled, W1, b1, W2, b2):
    BLK = 512
    return pl.pallas_call(
        _mlp_body,
        grid=(B // BLK,),
        in_specs=[
            pl.BlockSpec((BLK, EMB), lambda i: (i, 0)),
            pl.BlockSpec((EMB, HID), lambda i: (0, 0)),
            pl.BlockSpec((1, HID), lambda i: (0, 0)),
            pl.BlockSpec((HID, OUT), lambda i: (0, 0)),
            pl.BlockSpec((1, OUT), lambda i: (0, 0)),
        ],
        out_specs=pl.BlockSpec((BLK, OUT), lambda i: (i, 0)),
        out_shape=jax.ShapeDtypeStruct((B, OUT), jnp.float32),
    )(pooled, W1, b1.reshape(1, HID), W2, b2.reshape(1, OUT))


def kernel(x, table, W1, b1, W2, b2):
    pooled = _pool(x, table)
    return _mlp(pooled, W1, b1, W2, b2)
